# baseline (speedup 1.0000x reference)
import functools

import jax
import jax.numpy as jnp
from jax import lax
from jax.experimental import pallas as pl
from jax.experimental.pallas import tpu as pltpu
from jax.experimental.pallas import tpu_sc as plsc

N = 10000
D = 128
LANES = 16

NC = 2
NS = 16
NW = NC * NS

CHUNK = 64
CHUNKS_PER_WORKER = 160
SB = 8
NBLK = CHUNKS_PER_WORKER // SB
NBUF = 4
E_PAD = NW * CHUNKS_PER_WORKER * CHUNK
N_PAD = 10240
ROWS_PER_SUBCORE = N_PAD // NS


def _sc_aggregate(x_hbm, src_hbm, dst_hbm, a_hbm, zeros_hbm, out_hbm,
                  src_v, dst_v, a_v, rows_v, acc_sh, sem_i, sem_g, sem_s):
    c_ax = lax.axis_index("c")
    s_ax = lax.axis_index("s")
    wid = s_ax * NC + c_ax
    base = wid * CHUNKS_PER_WORKER

    pltpu.sync_copy(
        zeros_hbm,
        acc_sh.at[pl.ds(s_ax * ROWS_PER_SUBCORE, ROWS_PER_SUBCORE)])

    idx_pairs = ((src_hbm, src_v), (dst_hbm, dst_v), (a_hbm, a_v))

    def fire_idx(blk, ib):
        for ref_h, ref_v in idx_pairs:
            pltpu.async_copy(
                ref_h.at[pl.ds(base + blk * SB, SB)], ref_v.at[ib], sem_i)

    def wait_idx(blk, ib):
        for ref_h, ref_v in idx_pairs:
            pltpu.make_async_copy(
                ref_h.at[pl.ds(base + blk * SB, SB)], ref_v.at[ib],
                sem_i).wait()

    fire_idx(0, 0)
    wait_idx(0, 0)
    fire_idx(1, 1)
    pltpu.async_copy(x_hbm.at[src_v.at[0, 0]], rows_v.at[0], sem_g)
    pltpu.async_copy(x_hbm.at[src_v.at[0, 1]], rows_v.at[1], sem_g)
    plsc.subcore_barrier()

    def pipe_body(bi2, carry):
        for bb in range(2):
            for b in range(SB):
                ch = bi2 * (2 * SB) + bb * SB + b
                rb = b % NBUF

                pltpu.make_async_copy(
                    x_hbm.at[src_v.at[bb, b]], rows_v.at[rb], sem_g).wait()

                p_ib, p_r = (bb, b - 2) if b >= 2 else (1 - bb, SB - 2 + b)

                @pl.when(ch >= 2)
                def _():
                    pltpu.make_async_copy(
                        rows_v.at[(rb + 2) % NBUF],
                        acc_sh.at[dst_v.at[p_ib, p_r]], sem_s).wait()

                n_ib, n_r = (bb, b + 2) if b < SB - 2 else (1 - bb, b - 6)
                if b == SB - 2:
                    @pl.when(ch + 2 < CHUNKS_PER_WORKER)
                    def _():
                        wait_idx(bi2 * 2 + bb + 1, 1 - bb)
                        pltpu.async_copy(
                            x_hbm.at[src_v.at[n_ib, n_r]],
                            rows_v.at[(rb + 2) % NBUF], sem_g)
                elif b == SB - 1:
                    @pl.when(ch + 2 < CHUNKS_PER_WORKER)
                    def _():
                        pltpu.async_copy(
                            x_hbm.at[src_v.at[n_ib, n_r]],
                            rows_v.at[(rb + 2) % NBUF], sem_g)
                else:
                    pltpu.async_copy(
                        x_hbm.at[src_v.at[n_ib, n_r]],
                        rows_v.at[(rb + 2) % NBUF], sem_g)

                if b == 1:
                    blk = bi2 * 2 + bb

                    @pl.when((ch >= SB) & (ch < (NBLK - 1) * SB + 1))
                    def _():
                        fire_idx(blk + 1, 1 - bb)

                def t_body(t, carry2):
                    av16 = a_v[bb, b, pl.ds(t * LANES, LANES)]
                    for k in range(LANES):
                        e = t * LANES + k
                        av = jnp.full((LANES,), av16[k], jnp.float32)
                        for j2 in range(D // LANES):
                            sl = (rb, e, pl.ds(j2 * LANES, LANES))
                            rows_v[sl] = rows_v[sl] * av
                    return carry2
                lax.fori_loop(0, CHUNK // LANES, t_body, 0)

                pltpu.async_copy(
                    rows_v.at[rb], acc_sh.at[dst_v.at[bb, b]], sem_s,
                    add=True)
        return carry

    lax.fori_loop(0, NBLK // 2, pipe_body, 0)
    pltpu.make_async_copy(
        rows_v.at[(SB - 2) % NBUF],
        acc_sh.at[dst_v.at[1, SB - 2]], sem_s).wait()
    pltpu.make_async_copy(
        rows_v.at[(SB - 1) % NBUF],
        acc_sh.at[dst_v.at[1, SB - 1]], sem_s).wait()
    plsc.subcore_barrier()

    pltpu.sync_copy(
        acc_sh.at[pl.ds(s_ax * ROWS_PER_SUBCORE, ROWS_PER_SUBCORE)],
        out_hbm.at[c_ax, pl.ds(s_ax * ROWS_PER_SUBCORE, ROWS_PER_SUBCORE)])


_sc_agg_call = functools.partial(
    pl.kernel,
    out_type=jax.ShapeDtypeStruct((NC, N_PAD, D), jnp.float32),
    mesh=plsc.VectorSubcoreMesh(core_axis_name="c", subcore_axis_name="s"),
    scratch_types=[
        pltpu.VMEM((2, SB, CHUNK), jnp.int32),
        pltpu.VMEM((2, SB, CHUNK), jnp.int32),
        pltpu.VMEM((2, SB, CHUNK), jnp.float32),
        pltpu.VMEM((NBUF, CHUNK, D), jnp.float32),
        pltpu.VMEM_SHARED((N_PAD, D), jnp.float32),
        pltpu.SemaphoreType.DMA,
        pltpu.SemaphoreType.DMA,
        pltpu.SemaphoreType.DMA,
    ],
)(_sc_aggregate)


def _tc_finish(p0_ref, p1_ref, w_ref, o_ref):
    h = p0_ref[...] + p1_ref[...]
    o_ref[...] = jnp.maximum(
        jnp.dot(h, w_ref[...], preferred_element_type=jnp.float32), 0.0)


@jax.jit
def kernel(X, edge_index, A_vals, W):
    e = edge_index.shape[1]
    n_pad = E_PAD - e
    pad_idx = jnp.arange(n_pad, dtype=jnp.int32) % N
    src_p = jnp.concatenate([edge_index[0], pad_idx]).reshape(-1, CHUNK)
    dst_p = jnp.concatenate([edge_index[1], pad_idx]).reshape(-1, CHUNK)
    a_p = jnp.concatenate(
        [A_vals, jnp.zeros((n_pad,), jnp.float32)]).reshape(-1, CHUNK)
    zeros = jnp.zeros((ROWS_PER_SUBCORE, D), jnp.float32)

    partials = _sc_agg_call(X, src_p, dst_p, a_p, zeros)

    rows_blk = 1000
    out = pl.pallas_call(
        _tc_finish,
        grid=(N // rows_blk,),
        in_specs=[
            pl.BlockSpec((rows_blk, D), lambda i: (i, 0)),
            pl.BlockSpec((rows_blk, D), lambda i: (i, 0)),
            pl.BlockSpec((D, D), lambda i: (0, 0)),
        ],
        out_specs=pl.BlockSpec((rows_blk, D), lambda i: (i, 0)),
        out_shape=jax.ShapeDtypeStruct((N, D), jnp.float32),
    )(partials[0], partials[1], W)
    return out
